# baseline (speedup 1.0000x reference)
import functools

import jax
import jax.numpy as jnp
from jax import lax
from jax.experimental import pallas as pl
from jax.experimental.pallas import tpu as pltpu
from jax.experimental.pallas import tpu_sc as plsc

NC = 2
NS = 16
LANES = 16
NW = NC * NS
K = 80
SCH = 25
SUP = 5


def _sc_partials(x, ei5, w4):
    N, D = x.shape
    rows_per_tile = N // NS
    fgroups = D // LANES
    wb = rows_per_tile // 8 * 8
    wrem = N - NS * wb

    mesh = plsc.VectorSubcoreMesh(core_axis_name="c", subcore_axis_name="s")

    @functools.partial(
        pl.kernel,
        out_type=jax.ShapeDtypeStruct((NC, N, D), jnp.float32),
        mesh=mesh,
        scratch_types=[
            pltpu.VMEM((SCH, K), jnp.int32),
            pltpu.VMEM((SCH, K), jnp.int32),
            pltpu.VMEM((SCH, K), jnp.float32),
            pltpu.VMEM((2, K, D), jnp.float32),
            pltpu.VMEM_SHARED((N, D), jnp.float32),
            [pltpu.SemaphoreType.DMA] * 4,
            [pltpu.SemaphoreType.DMA] * 4,
        ],
    )
    def k(x_hbm, ei_hbm, w_hbm, part_hbm,
          src_v, dst_v, w_v, rows_v, acc_sh, sem0, sem1):
        c = lax.axis_index("c")
        s = lax.axis_index("s")
        wid = c * NS + s
        r0 = s * rows_per_tile
        sems = (sem0, sem1)
        H = K // 4

        zbuf = rows_v.at[0]

        def zrow(r, carry):
            for f in range(fgroups):
                zbuf[r, pl.ds(f * LANES, LANES)] = jnp.zeros(
                    (LANES,), jnp.float32)
            return carry

        lax.fori_loop(0, K, zrow, 0)
        full, rem = divmod(rows_per_tile, K)
        for j in range(full):
            pltpu.sync_copy(zbuf, acc_sh.at[pl.ds(r0 + j * K, K)])
        if rem:
            pltpu.sync_copy(zbuf.at[pl.ds(0, rem)],
                            acc_sh.at[pl.ds(r0 + full * K, rem)])
        plsc.subcore_barrier()

        def start_gather(t, b):
            for q in range(4):
                pltpu.async_copy(x_hbm.at[src_v.at[t, pl.ds(q * H, H)]],
                                 rows_v.at[b, pl.ds(q * H, H)], sems[b][q])

        def finish_chunk(t, b):
            for q in range(4):
                pltpu.make_async_copy(
                    x_hbm.at[src_v.at[t, pl.ds(q * H, H)]],
                    rows_v.at[b, pl.ds(q * H, H)], sems[b][q]).wait()
            buf = rows_v.at[b]

            def group(g, carry2):
                wg = w_v[t, pl.ds(g * LANES, LANES)]
                for i in range(LANES):
                    ws = wg.at[jnp.full((LANES,), i, jnp.int32)].get(
                        mode="promise_in_bounds")
                    e = g * LANES + i
                    for f in range(fgroups):
                        sl = pl.ds(f * LANES, LANES)
                        buf[e, sl] = buf[e, sl] * ws
                return carry2

            lax.fori_loop(0, K // LANES, group, 0)
            pltpu.sync_copy(buf, acc_sh.at[dst_v.at[t]], add=True)

        def superchunk(u, carry0):
            pltpu.sync_copy(ei_hbm.at[0, wid, u], src_v)
            pltpu.sync_copy(ei_hbm.at[1, wid, u], dst_v)
            pltpu.sync_copy(w_hbm.at[wid, u], w_v)
            start_gather(0, 0)

            def pair(p, carry):
                t0 = 2 * p
                start_gather(t0 + 1, 1)
                finish_chunk(t0, 0)

                @pl.when(t0 + 2 <= SCH - 1)
                def _():
                    start_gather(t0 + 2, 0)

                finish_chunk(t0 + 1, 1)
                return carry

            lax.fori_loop(0, SCH // 2, pair, 0)
            finish_chunk(SCH - 1, 0)
            return carry0

        lax.fori_loop(0, SUP, superchunk, 0)
        plsc.subcore_barrier()
        base = pl.multiple_of(s * wb, 8)
        pltpu.sync_copy(acc_sh.at[pl.ds(base, wb)],
                        part_hbm.at[c, pl.ds(base, wb)])
        if wrem:
            @pl.when(s == NS - 1)
            def _():
                pltpu.sync_copy(acc_sh.at[pl.ds(NS * wb, wrem)],
                                part_hbm.at[c, pl.ds(NS * wb, wrem)])

    return k(x, ei5, w4)


def _finish_tc(parts):
    ncp, N, D = parts.shape
    blk = 2000

    def body(p_ref, o_ref):
        h = p_ref[0] + p_ref[1]
        n2 = jnp.sum(h * h, axis=1, keepdims=True)
        o_ref[...] = h * lax.rsqrt(jnp.maximum(n2, 1e-24))

    return pl.pallas_call(
        body,
        grid=(N // blk,),
        in_specs=[pl.BlockSpec((ncp, blk, D), lambda i: (0, i, 0))],
        out_specs=pl.BlockSpec((blk, D), lambda i: (i, 0)),
        out_shape=jax.ShapeDtypeStruct((N, D), jnp.float32),
    )(parts)


def kernel(x, edge_index, edge_weight):
    ei5 = edge_index.reshape(2, NW, SUP, SCH, K)
    w4 = edge_weight.reshape(NW, SUP, SCH, K)
    parts = _sc_partials(x, ei5, w4)
    return _finish_tc(parts)


# speedup vs baseline: 1.0211x; 1.0211x over previous
import functools

import jax
import jax.numpy as jnp
from jax import lax
from jax.experimental import pallas as pl
from jax.experimental.pallas import tpu as pltpu
from jax.experimental.pallas import tpu_sc as plsc

NC = 2
NS = 16
LANES = 16
NW = NC * NS
K = 80
SCH = 25
SUP = 5


def _sc_partials(x, ei5, w4):
    N, D = x.shape
    rows_per_tile = N // NS
    fgroups = D // LANES
    wb = rows_per_tile // 8 * 8
    wrem = N - NS * wb

    mesh = plsc.VectorSubcoreMesh(core_axis_name="c", subcore_axis_name="s")

    @functools.partial(
        pl.kernel,
        out_type=jax.ShapeDtypeStruct((NC, N, D), jnp.float32),
        mesh=mesh,
        scratch_types=[
            pltpu.VMEM((SCH, K), jnp.int32),
            pltpu.VMEM((SCH, K), jnp.int32),
            pltpu.VMEM((SCH * K,), jnp.float32),
            pltpu.VMEM((2, K, D), jnp.float32),
            pltpu.VMEM_SHARED((N, D), jnp.float32),
            [pltpu.SemaphoreType.DMA] * 2,
            [pltpu.SemaphoreType.DMA] * 2,
        ],
    )
    def k(x_hbm, ei_hbm, w_hbm, part_hbm,
          src_v, dst_v, w_v, rows_v, acc_sh, sem0, sem1):
        c = lax.axis_index("c")
        s = lax.axis_index("s")
        wid = c * NS + s
        r0 = s * rows_per_tile
        sems = (sem0, sem1)
        H = K // 2

        zbuf = rows_v.at[0]

        def zrow(r, carry):
            for f in range(fgroups):
                zbuf[r, pl.ds(f * LANES, LANES)] = jnp.zeros(
                    (LANES,), jnp.float32)
            return carry

        lax.fori_loop(0, K, zrow, 0)
        full, rem = divmod(rows_per_tile, K)
        for j in range(full):
            pltpu.sync_copy(zbuf, acc_sh.at[pl.ds(r0 + j * K, K)])
        if rem:
            pltpu.sync_copy(zbuf.at[pl.ds(0, rem)],
                            acc_sh.at[pl.ds(r0 + full * K, rem)])
        plsc.subcore_barrier()

        def start_gather(t, b):
            for q in range(2):
                pltpu.async_copy(x_hbm.at[src_v.at[t, pl.ds(q * H, H)]],
                                 rows_v.at[b, pl.ds(q * H, H)], sems[b][q])

        def finish_chunk(t, b):
            for q in range(2):
                pltpu.make_async_copy(
                    x_hbm.at[src_v.at[t, pl.ds(q * H, H)]],
                    rows_v.at[b, pl.ds(q * H, H)], sems[b][q]).wait()
            buf = rows_v.at[b]

            def group(g, carry2):
                wg = w_v[pl.ds(t * K + g * LANES, LANES)]
                for i in range(LANES):
                    ws = wg.at[jnp.full((LANES,), i, jnp.int32)].get(
                        mode="promise_in_bounds")
                    e = g * LANES + i
                    for f in range(fgroups):
                        sl = pl.ds(f * LANES, LANES)
                        buf[e, sl] = buf[e, sl] * ws
                return carry2

            lax.fori_loop(0, K // LANES, group, 0)
            pltpu.sync_copy(buf, acc_sh.at[dst_v.at[t]], add=True)

        def superchunk(u, carry0):
            pltpu.sync_copy(ei_hbm.at[0, wid, u], src_v)
            pltpu.sync_copy(ei_hbm.at[1, wid, u], dst_v)
            pltpu.sync_copy(
                w_hbm.at[pl.ds(wid * (SUP * SCH * K) + u * (SCH * K),
                               SCH * K)], w_v)
            start_gather(0, 0)

            def pair(p, carry):
                t0 = 2 * p
                start_gather(t0 + 1, 1)
                finish_chunk(t0, 0)

                @pl.when(t0 + 2 <= SCH - 1)
                def _():
                    start_gather(t0 + 2, 0)

                finish_chunk(t0 + 1, 1)
                return carry

            lax.fori_loop(0, SCH // 2, pair, 0)
            finish_chunk(SCH - 1, 0)
            return carry0

        lax.fori_loop(0, SUP, superchunk, 0)
        plsc.subcore_barrier()
        base = pl.multiple_of(s * wb, 8)
        pltpu.sync_copy(acc_sh.at[pl.ds(base, wb)],
                        part_hbm.at[c, pl.ds(base, wb)])
        if wrem:
            @pl.when(s == NS - 1)
            def _():
                pltpu.sync_copy(acc_sh.at[pl.ds(NS * wb, wrem)],
                                part_hbm.at[c, pl.ds(NS * wb, wrem)])

    return k(x, ei5, w4)


def _finish_tc(parts):
    ncp, N, D = parts.shape
    blk = 2000

    def body(p_ref, o_ref):
        h = p_ref[0] + p_ref[1]
        n2 = jnp.sum(h * h, axis=1, keepdims=True)
        o_ref[...] = h * lax.rsqrt(jnp.maximum(n2, 1e-24))

    return pl.pallas_call(
        body,
        grid=(N // blk,),
        in_specs=[pl.BlockSpec((ncp, blk, D), lambda i: (0, i, 0))],
        out_specs=pl.BlockSpec((blk, D), lambda i: (i, 0)),
        out_shape=jax.ShapeDtypeStruct((N, D), jnp.float32),
    )(parts)


def kernel(x, edge_index, edge_weight):
    ei5 = edge_index.reshape(2, NW, SUP, SCH, K)
    parts = _sc_partials(x, ei5, edge_weight)
    return _finish_tc(parts)
